# Initial kernel scaffold; baseline (speedup 1.0000x reference)
import jax
import jax.numpy as jnp
from jax.experimental import pallas as pl


def kernel(inputs, indices, output_shape):
    raise NotImplementedError("write your pallas kernel here")


# speedup vs baseline: 16.1416x; 16.1416x over previous
import jax
import jax.numpy as jnp
from jax import lax
from jax.experimental import pallas as pl
from jax.experimental.pallas import tpu as pltpu
from jax.experimental.pallas import tpu_sc as plsc

B, H, W, C = 4, 112, 112, 96
OH, OW = 224, 224
N = H * W * C
F = OH * OW * C
NC, NS = 2, 16
NPASS = 4
CHUNK = F // NPASS
PT = CHUNK // NS
NT = N // NS
KBLK = 4704
NBLK = NT // KBLK
INNER = KBLK // 16
ZBUF = 10752
NZ = PT // ZBUF
DSPREAD = 64
ACCPAD = NS * DSPREAD


def _sc_unpool_kernel(idx_hbm, val_hbm, out_hbm,
                      in_idx, in_val, st_idx, st_val, zbuf, acc):
    cid = lax.axis_index("c")
    sid = lax.axis_index("s")

    zeros16 = jnp.zeros((16,), jnp.float32)

    def zero_zbuf(i, carry):
        zbuf[pl.ds(i * 16, 16)] = zeros16
        return carry

    lax.fori_loop(0, ZBUF // 16, zero_zbuf, 0)

    dump_base = CHUNK + sid * DSPREAD

    for b_local in range(B // NC):
        b = cid * (B // NC) + b_local
        in_base = b * N + sid * NT
        for p in range(NPASS):
            lo = p * CHUNK

            def zero_slice(j, carry):
                pltpu.sync_copy(zbuf, acc.at[pl.ds(sid * PT + j * ZBUF, ZBUF)])
                return carry

            lax.fori_loop(0, NZ, zero_slice, 0)
            plsc.subcore_barrier()

            def block(jb, carry):
                pltpu.sync_copy(idx_hbm.at[pl.ds(in_base + jb * KBLK, KBLK)],
                                in_idx)
                pltpu.sync_copy(val_hbm.at[pl.ds(in_base + jb * KBLK, KBLK)],
                                in_val)

                def body(i, c2):
                    ii = i * 16
                    idx = in_idx[pl.ds(ii, 16)]
                    v = in_val[pl.ds(ii, 16)]
                    rel = idx - lo
                    inb = (rel >= 0) & (rel < CHUNK)
                    st_idx[pl.ds(ii, 16)] = jnp.where(
                        inb, rel, dump_base + (idx & (DSPREAD - 1)))
                    st_val[pl.ds(ii, 16)] = jnp.where(inb, v, 0.0)
                    return c2

                lax.fori_loop(0, INNER, body, 0)
                pltpu.sync_copy(st_val, acc.at[st_idx], add=True)
                return carry

            lax.fori_loop(0, NBLK, block, 0)
            plsc.subcore_barrier()

            pltpu.sync_copy(
                acc.at[pl.ds(sid * PT, PT)],
                out_hbm.at[pl.ds(b * F + lo + sid * PT, PT)])


def kernel(inputs, indices, output_shape):
    del output_shape
    idx_flat = indices.reshape(-1)
    val_flat = inputs.reshape(-1)
    mesh = plsc.VectorSubcoreMesh(core_axis_name="c", subcore_axis_name="s")
    out = pl.kernel(
        _sc_unpool_kernel,
        out_type=jax.ShapeDtypeStruct((B * F,), jnp.float32),
        mesh=mesh,
        scratch_types=[
            pltpu.VMEM((KBLK,), jnp.int32),
            pltpu.VMEM((KBLK,), jnp.float32),
            pltpu.VMEM((KBLK,), jnp.int32),
            pltpu.VMEM((KBLK,), jnp.float32),
            pltpu.VMEM((ZBUF,), jnp.float32),
            pltpu.VMEM_SHARED((CHUNK + ACCPAD,), jnp.float32),
        ],
    )(idx_flat, val_flat)
    return out.reshape(B, OH, OW, C)


# speedup vs baseline: 25.8985x; 1.6045x over previous
import jax
import jax.numpy as jnp
from jax import lax
from jax.experimental import pallas as pl
from jax.experimental.pallas import tpu as pltpu
from jax.experimental.pallas import tpu_sc as plsc

B, H, W, C = 4, 112, 112, 96
OH, OW = 224, 224
N = H * W * C
F = OH * OW * C
NC, NS = 2, 16
NPASS = 4
CHUNK = F // NPASS
PT = CHUNK // NS
NT = N // NS
KBLK = 3136
NBLK = NT // KBLK
INNER = KBLK // 16
ZBUF = 10752
NZ = PT // ZBUF
DSPREAD = 64
ACCPAD = NS * DSPREAD


def _sc_unpool_kernel(idx_hbm, val_hbm, out_hbm,
                      in_idx0, in_val0, in_idx1, in_val1,
                      st_idx0, st_val0, st_idx1, st_val1,
                      zbuf, acc, ld_sem0, ld_sem1, sct_sem0, sct_sem1):
    cid = lax.axis_index("c")
    sid = lax.axis_index("s")

    zeros16 = jnp.zeros((16,), jnp.float32)

    def zero_zbuf(i, carry):
        zbuf[pl.ds(i * 16, 16)] = zeros16
        return carry

    lax.fori_loop(0, ZBUF // 16, zero_zbuf, 0)

    dump_base = CHUNK + sid * DSPREAD

    ins = ((in_idx0, in_val0, ld_sem0), (in_idx1, in_val1, ld_sem1))
    sts = ((st_idx0, st_val0, sct_sem0), (st_idx1, st_val1, sct_sem1))

    for b_local in range(B // NC):
        b = cid * (B // NC) + b_local
        in_base = b * N + sid * NT
        for p in range(NPASS):
            lo = p * CHUNK

            def zero_slice(j, carry):
                pltpu.sync_copy(zbuf, acc.at[pl.ds(sid * PT + j * ZBUF, ZBUF)])
                return carry

            pltpu.async_copy(idx_hbm.at[pl.ds(in_base, KBLK)], in_idx0,
                             ld_sem0)
            pltpu.async_copy(val_hbm.at[pl.ds(in_base, KBLK)], in_val0,
                             ld_sem0)
            lax.fori_loop(0, NZ, zero_slice, 0)
            plsc.subcore_barrier()

            def pair(jj, carry):
                for par in range(2):
                    jb = jj * 2 + par
                    ci, cv, ld_sem = ins[par]
                    si, sv, sct_sem = sts[par]
                    ni, nv, nld_sem = ins[1 - par]

                    @pl.when(jb + 1 < NBLK)
                    def _():
                        off = in_base + (jb + 1) * KBLK
                        pltpu.async_copy(idx_hbm.at[pl.ds(off, KBLK)], ni,
                                         nld_sem)
                        pltpu.async_copy(val_hbm.at[pl.ds(off, KBLK)], nv,
                                         nld_sem)

                    pltpu.make_async_copy(
                        idx_hbm.at[pl.ds(in_base, KBLK)], ci, ld_sem).wait()
                    pltpu.make_async_copy(
                        val_hbm.at[pl.ds(in_base, KBLK)], cv, ld_sem).wait()

                    @pl.when(jb >= 2)
                    def _():
                        pltpu.make_async_copy(sv, acc.at[si], sct_sem).wait()

                    def group(i, c2):
                        ii = i * 16
                        idx = ci[pl.ds(ii, 16)]
                        v = cv[pl.ds(ii, 16)]
                        rel = idx - lo
                        inb = plsc.bitcast(rel, jnp.uint32) < jnp.uint32(CHUNK)
                        si[pl.ds(ii, 16)] = jnp.where(
                            inb, rel, dump_base + (idx & (DSPREAD - 1)))
                        sv[pl.ds(ii, 16)] = jnp.where(inb, v, 0.0)
                        return c2

                    lax.fori_loop(0, INNER, group, 0)
                    pltpu.async_copy(sv, acc.at[si], sct_sem, add=True)
                return carry

            lax.fori_loop(0, NBLK // 2, pair, 0)

            pltpu.make_async_copy(st_val0, acc.at[st_idx0], sct_sem0).wait()
            pltpu.make_async_copy(st_val1, acc.at[st_idx1], sct_sem1).wait()
            plsc.subcore_barrier()

            pltpu.sync_copy(
                acc.at[pl.ds(sid * PT, PT)],
                out_hbm.at[pl.ds(b * F + lo + sid * PT, PT)])


def kernel(inputs, indices, output_shape):
    del output_shape
    idx_flat = indices.reshape(-1)
    val_flat = inputs.reshape(-1)
    mesh = plsc.VectorSubcoreMesh(core_axis_name="c", subcore_axis_name="s")
    out = pl.kernel(
        _sc_unpool_kernel,
        out_type=jax.ShapeDtypeStruct((B * F,), jnp.float32),
        mesh=mesh,
        scratch_types=[
            pltpu.VMEM((KBLK,), jnp.int32),
            pltpu.VMEM((KBLK,), jnp.float32),
            pltpu.VMEM((KBLK,), jnp.int32),
            pltpu.VMEM((KBLK,), jnp.float32),
            pltpu.VMEM((KBLK,), jnp.int32),
            pltpu.VMEM((KBLK,), jnp.float32),
            pltpu.VMEM((KBLK,), jnp.int32),
            pltpu.VMEM((KBLK,), jnp.float32),
            pltpu.VMEM((ZBUF,), jnp.float32),
            pltpu.VMEM_SHARED((CHUNK + ACCPAD,), jnp.float32),
            pltpu.SemaphoreType.DMA,
            pltpu.SemaphoreType.DMA,
            pltpu.SemaphoreType.DMA,
            pltpu.SemaphoreType.DMA,
        ],
    )(idx_flat, val_flat)
    return out.reshape(B, OH, OW, C)
